# baseline (speedup 1.0000x reference)
import functools

import jax
import jax.numpy as jnp
from jax import lax
from jax.experimental import pallas as pl
from jax.experimental.pallas import tpu as pltpu


def _mha_kernel(x_ref, w_ref, bqkv_ref, wo_ref, ob_ref,
                out_ref, attn_ref, ctx_s_ref, *, g, dk):
    j = pl.program_id(1)
    gd = g * dk
    D = x_ref.shape[2]

    x = x_ref[0].astype(jnp.bfloat16)

    def proj_group(base):
        acc = jnp.dot(x, w_ref[:, pl.ds(base, gd)],
                      preferred_element_type=jnp.float32)
        return (acc + bqkv_ref[:, pl.ds(base, gd)]).astype(jnp.bfloat16)

    q = proj_group(j * gd)
    k = proj_group(D + j * gd)

    def qk(h):
        sl = slice(h * dk, (h + 1) * dk)
        return lax.dot_general(q[:, sl], k[:, sl], (((1,), (1,)), ((), ())),
                               preferred_element_type=jnp.float32)

    ctx_parts = []
    s = qk(0)
    for h in range(g):
        row_max = jnp.max(s, axis=-1, keepdims=True)
        s_cur = s
        if h == 0:
            v = proj_group(2 * D + j * gd)
        if h + 1 < g:
            s = qk(h + 1)
        eb = jnp.exp(s_cur - row_max).astype(jnp.bfloat16)
        ctx_un = jnp.dot(eb, v[:, h * dk:(h + 1) * dk],
                         preferred_element_type=jnp.float32)
        denom = jnp.sum(eb.astype(jnp.float32), axis=-1, keepdims=True)
        recip = 1.0 / denom
        attn_ref[0, h] = eb.astype(jnp.float32) * recip
        ctx_parts.append(ctx_un * recip)

    ctx = jnp.concatenate(ctx_parts, axis=1).astype(jnp.bfloat16)

    @pl.when(j == 0)
    def _stage():
        ctx_s_ref[...] = ctx

    @pl.when(j != 0)
    def _project_out():
        ctx_full = jnp.concatenate([ctx_s_ref[...], ctx], axis=1)
        out_ref[0] = (jnp.dot(ctx_full, wo_ref[...],
                              preferred_element_type=jnp.float32)
                      + ob_ref[...])


def kernel(x, qkv_wt, qkv_b, o_wt, o_b):
    bs, L, D = x.shape
    dk = 64
    nh = D // dk
    g = 8
    nj = nh // g
    gd = g * dk

    b2 = qkv_b.reshape(1, 3 * D).astype(jnp.float32)
    ob2 = o_b.reshape(1, D).astype(jnp.float32)

    out, attn = pl.pallas_call(
        functools.partial(_mha_kernel, g=g, dk=dk),
        out_shape=(
            jax.ShapeDtypeStruct((bs, L, D), jnp.float32),
            jax.ShapeDtypeStruct((bs, nh, L, L), jnp.float32),
        ),
        grid=(bs, nj),
        in_specs=[
            pl.BlockSpec((1, L, D), lambda b, j: (b, 0, 0)),
            pl.BlockSpec((D, 3 * D), lambda b, j: (0, 0)),
            pl.BlockSpec((1, 3 * D), lambda b, j: (0, 0)),
            pl.BlockSpec((D, D), lambda b, j: (0, 0)),
            pl.BlockSpec((1, D), lambda b, j: (0, 0)),
        ],
        out_specs=(
            pl.BlockSpec((1, L, D), lambda b, j: (b, 0, 0)),
            pl.BlockSpec((1, g, L, L), lambda b, j: (b, j, 0, 0)),
        ),
        scratch_shapes=[pltpu.VMEM((L, gd), jnp.bfloat16)],
        compiler_params=pltpu.CompilerParams(
            dimension_semantics=("parallel", "arbitrary"),
            vmem_limit_bytes=56 * 1024 * 1024,
        ),
    )(x, qkv_wt, b2, o_wt, ob2)
    return out, attn


# speedup vs baseline: 1.0040x; 1.0040x over previous
import functools

import jax
import jax.numpy as jnp
from jax import lax
from jax.experimental import pallas as pl
from jax.experimental.pallas import tpu as pltpu


def _mha_kernel(x_ref, w_ref, bqkv_ref, wo_ref, ob_ref,
                out_ref, attn_ref, ctx_s_ref, *, g, dk):
    j = pl.program_id(1)
    gd = g * dk
    D = x_ref.shape[2]

    x = x_ref[0].astype(jnp.bfloat16)

    def proj_group(base):
        acc = jnp.dot(x, w_ref[:, pl.ds(base, gd)],
                      preferred_element_type=jnp.float32)
        return (acc + bqkv_ref[:, pl.ds(base, gd)]).astype(jnp.bfloat16)

    q = proj_group(j * gd)
    k = proj_group(D + j * gd)
    v = proj_group(2 * D + j * gd)

    def qk(h):
        sl = slice(h * dk, (h + 1) * dk)
        return lax.dot_general(q[:, sl], k[:, sl], (((1,), (1,)), ((), ())),
                               preferred_element_type=jnp.float32)

    ctx_parts = []
    s = qk(0)
    for h in range(g):
        row_max = jnp.max(s, axis=-1, keepdims=True)
        eb = jnp.exp(s - row_max).astype(jnp.bfloat16)
        if h + 1 < g:
            s = qk(h + 1)
        denom = jnp.sum(eb.astype(jnp.float32), axis=-1, keepdims=True)
        recip = 1.0 / denom
        attn_ref[0, h] = eb.astype(jnp.float32) * recip
        ctx_un = jnp.dot(eb, v[:, h * dk:(h + 1) * dk],
                         preferred_element_type=jnp.float32)
        ctx_parts.append(ctx_un * recip)

    ctx = jnp.concatenate(ctx_parts, axis=1).astype(jnp.bfloat16)

    @pl.when(j == 0)
    def _stage():
        ctx_s_ref[...] = ctx

    @pl.when(j != 0)
    def _project_out():
        ctx_full = jnp.concatenate([ctx_s_ref[...], ctx], axis=1)
        out_ref[0] = (jnp.dot(ctx_full, wo_ref[...],
                              preferred_element_type=jnp.float32)
                      + ob_ref[...])


def kernel(x, qkv_wt, qkv_b, o_wt, o_b):
    bs, L, D = x.shape
    dk = 64
    nh = D // dk
    g = 8
    nj = nh // g
    gd = g * dk

    b2 = qkv_b.reshape(1, 3 * D).astype(jnp.float32)
    ob2 = o_b.reshape(1, D).astype(jnp.float32)

    out, attn = pl.pallas_call(
        functools.partial(_mha_kernel, g=g, dk=dk),
        out_shape=(
            jax.ShapeDtypeStruct((bs, L, D), jnp.float32),
            jax.ShapeDtypeStruct((bs, nh, L, L), jnp.float32),
        ),
        grid=(bs, nj),
        in_specs=[
            pl.BlockSpec((1, L, D), lambda b, j: (b, 0, 0)),
            pl.BlockSpec((D, 3 * D), lambda b, j: (0, 0)),
            pl.BlockSpec((1, 3 * D), lambda b, j: (0, 0)),
            pl.BlockSpec((D, D), lambda b, j: (0, 0)),
            pl.BlockSpec((1, D), lambda b, j: (0, 0)),
        ],
        out_specs=(
            pl.BlockSpec((1, L, D), lambda b, j: (b, 0, 0)),
            pl.BlockSpec((1, g, L, L), lambda b, j: (b, j, 0, 0)),
        ),
        scratch_shapes=[pltpu.VMEM((L, gd), jnp.bfloat16)],
        compiler_params=pltpu.CompilerParams(
            dimension_semantics=("parallel", "arbitrary"),
            vmem_limit_bytes=56 * 1024 * 1024,
        ),
    )(x, qkv_wt, b2, o_wt, ob2)
    return out, attn


# speedup vs baseline: 1.0050x; 1.0010x over previous
import functools

import jax
import jax.numpy as jnp
from jax import lax
from jax.experimental import pallas as pl
from jax.experimental.pallas import tpu as pltpu


def _mha_kernel(x_ref, w_ref, bqkv_ref, wo_ref, ob_ref,
                out_ref, attn_ref, ctx_s_ref, *, g, dk):
    j = pl.program_id(1)
    gd = g * dk
    D = x_ref.shape[2]

    x = x_ref[0].astype(jnp.bfloat16)

    def proj_group(base):
        acc = jnp.dot(x, w_ref[:, pl.ds(base, gd)],
                      preferred_element_type=jnp.float32)
        return (acc + bqkv_ref[:, pl.ds(base, gd)]).astype(jnp.bfloat16)

    hw = gd // 2
    hg = g // 2

    def proj_half(base):
        acc = jnp.dot(x, w_ref[:, pl.ds(base, hw)],
                      preferred_element_type=jnp.float32)
        return (acc + bqkv_ref[:, pl.ds(base, hw)]).astype(jnp.bfloat16)

    halves = {"q": [proj_half(j * gd), None],
              "k": [proj_half(D + j * gd), None],
              "v": [proj_half(2 * D + j * gd), None]}
    late = [("q", j * gd + hw), ("k", D + j * gd + hw),
            ("v", 2 * D + j * gd + hw)]

    def qk(h):
        c, hh = divmod(h, hg)
        sl = slice(hh * dk, (hh + 1) * dk)
        qc = halves["q"][c]
        kc = halves["k"][c]
        return lax.dot_general(qc[:, sl], kc[:, sl], (((1,), (1,)), ((), ())),
                               preferred_element_type=jnp.float32)

    ctx_parts = []
    s = qk(0)
    for h in range(g):
        row_max = jnp.max(s, axis=-1, keepdims=True)
        eb = jnp.exp(s - row_max).astype(jnp.bfloat16)
        if h < len(late):
            name, base = late[h]
            halves[name][1] = proj_half(base)
        if h + 1 < g:
            s = qk(h + 1)
        denom = jnp.sum(eb.astype(jnp.float32), axis=-1, keepdims=True)
        recip = 1.0 / denom
        attn_ref[0, h] = eb.astype(jnp.float32) * recip
        c, hh = divmod(h, hg)
        ctx_un = jnp.dot(eb, halves["v"][c][:, hh * dk:(hh + 1) * dk],
                         preferred_element_type=jnp.float32)
        ctx_parts.append(ctx_un * recip)

    ctx = jnp.concatenate(ctx_parts, axis=1).astype(jnp.bfloat16)

    @pl.when(j == 0)
    def _stage():
        ctx_s_ref[...] = ctx

    @pl.when(j != 0)
    def _project_out():
        ctx_full = jnp.concatenate([ctx_s_ref[...], ctx], axis=1)
        out_ref[0] = (jnp.dot(ctx_full, wo_ref[...],
                              preferred_element_type=jnp.float32)
                      + ob_ref[...])


def kernel(x, qkv_wt, qkv_b, o_wt, o_b):
    bs, L, D = x.shape
    dk = 64
    nh = D // dk
    g = 8
    nj = nh // g
    gd = g * dk

    b2 = qkv_b.reshape(1, 3 * D).astype(jnp.float32)
    ob2 = o_b.reshape(1, D).astype(jnp.float32)

    out, attn = pl.pallas_call(
        functools.partial(_mha_kernel, g=g, dk=dk),
        out_shape=(
            jax.ShapeDtypeStruct((bs, L, D), jnp.float32),
            jax.ShapeDtypeStruct((bs, nh, L, L), jnp.float32),
        ),
        grid=(bs, nj),
        in_specs=[
            pl.BlockSpec((1, L, D), lambda b, j: (b, 0, 0)),
            pl.BlockSpec((D, 3 * D), lambda b, j: (0, 0)),
            pl.BlockSpec((1, 3 * D), lambda b, j: (0, 0)),
            pl.BlockSpec((D, D), lambda b, j: (0, 0)),
            pl.BlockSpec((1, D), lambda b, j: (0, 0)),
        ],
        out_specs=(
            pl.BlockSpec((1, L, D), lambda b, j: (b, 0, 0)),
            pl.BlockSpec((1, g, L, L), lambda b, j: (b, j, 0, 0)),
        ),
        scratch_shapes=[pltpu.VMEM((L, gd), jnp.bfloat16)],
        compiler_params=pltpu.CompilerParams(
            dimension_semantics=("parallel", "arbitrary"),
            vmem_limit_bytes=56 * 1024 * 1024,
        ),
    )(x, qkv_wt, b2, o_wt, ob2)
    return out, attn
